# baseline (speedup 1.0000x reference)
import functools

import jax
import jax.numpy as jnp
from jax import lax
from jax.experimental import pallas as pl
from jax.experimental.pallas import tpu as pltpu
from jax.experimental.pallas import tpu_sc as plsc

NCLS = 10000
BATCH = 4096
DIM = 512
MIX_ALPHA = 0.2

NSUB = 16
NWORKERS = 32
ROWS_PER_W = BATCH // NWORKERS
ZWORDS = ROWS_PER_W * NCLS
ZSLICE = ZWORDS // NSUB
LANES = 16
NGRP = ROWS_PER_W // LANES

TC_BLK = 128


def _tc_mix_body(lam_ref, p_ref, a_ref, b_ref, o_ref):
    lam = lam_ref[0, 0]
    rev = jnp.dot(p_ref[...], b_ref[...], preferred_element_type=jnp.float32)
    o_ref[...] = a_ref[...] * lam + rev * (1.0 - lam)


_sc_mesh = plsc.VectorSubcoreMesh(core_axis_name="c", subcore_axis_name="s")


@functools.partial(
    pl.kernel,
    mesh=_sc_mesh,
    compiler_params=pltpu.CompilerParams(needs_layout_passes=False),
    out_type=jax.ShapeDtypeStruct((BATCH * NCLS,), jnp.float32),
    scratch_types=[
        pltpu.VMEM((ROWS_PER_W,), jnp.int32),
        pltpu.VMEM((ROWS_PER_W,), jnp.int32),
        pltpu.VMEM((3 * LANES,), jnp.float32),
        pltpu.VMEM((2, ROWS_PER_W), jnp.int32),
        pltpu.VMEM((2, ROWS_PER_W), jnp.float32),
        pltpu.VMEM_SHARED((ZWORDS,), jnp.float32),
    ],
)
def _sc_targets(tgt_hbm, vals_hbm, zeros_hbm, out_hbm,
                tgt_v, rev_v, vals_v, sidx_v, sval_v, zblock):
    cid = lax.axis_index("c")
    sid = lax.axis_index("s")
    wid = sid * 2 + cid
    base = wid * ROWS_PER_W

    pltpu.sync_copy(tgt_hbm.at[pl.ds(base, ROWS_PER_W)], tgt_v)
    pltpu.sync_copy(
        tgt_hbm.at[pl.ds(BATCH - base - ROWS_PER_W, ROWS_PER_W)], rev_v)
    pltpu.sync_copy(vals_hbm, vals_v)

    pltpu.sync_copy(zeros_hbm.at[pl.ds(sid * ZSLICE, ZSLICE)],
                    zblock.at[pl.ds(sid * ZSLICE, ZSLICE)])

    jlane = lax.iota(jnp.int32, 16)
    lam_vec = vals_v[pl.ds(0, LANES)]
    lamc_vec = vals_v[pl.ds(LANES, LANES)]
    one_vec = vals_v[pl.ds(2 * LANES, LANES)]
    for g in range(NGRP):
        rloc = g * LANES + jlane
        ca = plsc.load_gather(tgt_v, [rloc])
        cb = plsc.load_gather(rev_v, [(ROWS_PER_W - 1) - rloc])
        coll = ca == cb
        rowoff = (base + rloc) * NCLS
        sidx_v[0, pl.ds(g * LANES, LANES)] = rowoff + ca
        sidx_v[1, pl.ds(g * LANES, LANES)] = rowoff + cb
        sval_v[0, pl.ds(g * LANES, LANES)] = jnp.where(coll, one_vec, lam_vec)
        sval_v[1, pl.ds(g * LANES, LANES)] = jnp.where(coll, one_vec, lamc_vec)

    plsc.subcore_barrier()

    pltpu.sync_copy(zblock, out_hbm.at[pl.ds(base * NCLS, ZWORDS)])

    pltpu.sync_copy(sval_v.at[0], out_hbm.at[sidx_v.at[0]])
    pltpu.sync_copy(sval_v.at[1], out_hbm.at[sidx_v.at[1]])


def kernel(inputs, targets):
    lam = jax.random.beta(jax.random.key(42), MIX_ALPHA, MIX_ALPHA)
    lam = lam.astype(jnp.float32)
    lamc = 1.0 - lam

    nblk = BATCH // TC_BLK
    perm = jnp.flipud(jnp.eye(TC_BLK, dtype=jnp.float32))
    inputs_mixed = pl.pallas_call(
        _tc_mix_body,
        grid=(nblk,),
        in_specs=[
            pl.BlockSpec((1, 1), lambda i: (0, 0)),
            pl.BlockSpec((TC_BLK, TC_BLK), lambda i: (0, 0)),
            pl.BlockSpec((TC_BLK, DIM), lambda i: (i, 0)),
            pl.BlockSpec((TC_BLK, DIM), lambda i: (nblk - 1 - i, 0)),
        ],
        out_specs=pl.BlockSpec((TC_BLK, DIM), lambda i: (i, 0)),
        out_shape=jax.ShapeDtypeStruct((BATCH, DIM), jnp.float32),
    )(lam.reshape(1, 1), perm, inputs, inputs)

    vals = jnp.concatenate([
        jnp.full((LANES,), lam, jnp.float32),
        jnp.full((LANES,), lamc, jnp.float32),
        jnp.full((LANES,), lam + lamc, jnp.float32),
    ])
    zeros_hbm = jnp.zeros((ZWORDS,), jnp.float32)
    targets_mixed = _sc_targets(
        targets, vals, zeros_hbm).reshape(BATCH, NCLS)

    return (inputs_mixed, targets_mixed)


# speedup vs baseline: 1.3856x; 1.3856x over previous
import jax
import jax.numpy as jnp
from jax import lax
from jax.experimental import pallas as pl
from jax.experimental.pallas import tpu as pltpu

NCLS = 10000
BATCH = 4096
DIM = 512
MIX_ALPHA = 0.2

TC_BLK = 128
TROWS = 128


def _tc_mix_body(lam_ref, p_ref, a_ref, b_ref, o_ref):
    lam = lam_ref[0, 0]
    rev = jnp.dot(p_ref[...], b_ref[...], preferred_element_type=jnp.float32)
    o_ref[...] = a_ref[...] * lam + rev * (1.0 - lam)


def _tc_targets_body(lam_ref, p_ref, t1_ref, t2_ref, o_ref):
    lam = lam_ref[0, 0]
    lamc = 1.0 - lam
    t1 = t1_ref[...]
    t2 = jnp.dot(p_ref[...], t2_ref[...],
                 preferred_element_type=jnp.float32)
    c = lax.broadcasted_iota(jnp.int32, (TROWS, NCLS), 1).astype(jnp.float32)
    zero = jnp.zeros((), jnp.float32)
    o_ref[...] = (jnp.where(c == t1, lam, zero)
                  + jnp.where(jnp.abs(c - t2) < 0.5, lamc, zero))


def kernel(inputs, targets):
    lam = jax.random.beta(jax.random.key(42), MIX_ALPHA, MIX_ALPHA)
    lam = lam.astype(jnp.float32)

    nblk = BATCH // TC_BLK
    perm = jnp.flipud(jnp.eye(TC_BLK, dtype=jnp.float32))
    inputs_mixed = pl.pallas_call(
        _tc_mix_body,
        grid=(nblk,),
        in_specs=[
            pl.BlockSpec((1, 1), lambda i: (0, 0)),
            pl.BlockSpec((TC_BLK, TC_BLK), lambda i: (0, 0)),
            pl.BlockSpec((TC_BLK, DIM), lambda i: (i, 0)),
            pl.BlockSpec((TC_BLK, DIM), lambda i: (nblk - 1 - i, 0)),
        ],
        out_specs=pl.BlockSpec((TC_BLK, DIM), lambda i: (i, 0)),
        out_shape=jax.ShapeDtypeStruct((BATCH, DIM), jnp.float32),
    )(lam.reshape(1, 1), perm, inputs, inputs)

    tgt_f = targets.astype(jnp.float32).reshape(BATCH, 1)
    tblk = BATCH // TROWS
    permt = jnp.flipud(jnp.eye(TROWS, dtype=jnp.float32))
    targets_mixed = pl.pallas_call(
        _tc_targets_body,
        grid=(tblk,),
        in_specs=[
            pl.BlockSpec((1, 1), lambda i: (0, 0)),
            pl.BlockSpec((TROWS, TROWS), lambda i: (0, 0)),
            pl.BlockSpec((TROWS, 1), lambda i: (i, 0)),
            pl.BlockSpec((TROWS, 1), lambda i: (tblk - 1 - i, 0)),
        ],
        out_specs=pl.BlockSpec((TROWS, NCLS), lambda i: (i, 0)),
        out_shape=jax.ShapeDtypeStruct((BATCH, NCLS), jnp.float32),
    )(lam.reshape(1, 1), permt, tgt_f, tgt_f)

    return (inputs_mixed, targets_mixed)


# speedup vs baseline: 1.4294x; 1.0316x over previous
import jax
import jax.numpy as jnp
from jax import lax
from jax.experimental import pallas as pl
from jax.experimental.pallas import tpu as pltpu

NCLS = 10000
BATCH = 4096
DIM = 512
MIX_ALPHA = 0.2

TC_BLK = 128
TROWS = 128


def _tc_mix_body(lam_ref, p_ref, a_ref, b_ref, o_ref):
    lam = lam_ref[0, 0]
    rev = jnp.dot(p_ref[...], b_ref[...], preferred_element_type=jnp.float32)
    o_ref[...] = a_ref[...] * lam + rev * (1.0 - lam)


def _tc_targets_body(lam_ref, t1_ref, t2_ref, o_ref):
    lam = lam_ref[0, 0]
    lamc = 1.0 - lam
    t1 = t1_ref[...]
    t2 = t2_ref[...]
    c = lax.broadcasted_iota(jnp.int32, (TROWS, NCLS), 1)
    zero = jnp.zeros((), jnp.float32)
    o_ref[...] = (jnp.where(c == t1, lam, zero)
                  + jnp.where(c == t2, lamc, zero))


def kernel(inputs, targets):
    lam = jax.random.beta(jax.random.key(42), MIX_ALPHA, MIX_ALPHA)
    lam = lam.astype(jnp.float32)

    nblk = BATCH // TC_BLK
    perm = jnp.flipud(jnp.eye(TC_BLK, dtype=jnp.float32))
    inputs_mixed = pl.pallas_call(
        _tc_mix_body,
        grid=(nblk,),
        in_specs=[
            pl.BlockSpec((1, 1), lambda i: (0, 0)),
            pl.BlockSpec((TC_BLK, TC_BLK), lambda i: (0, 0)),
            pl.BlockSpec((TC_BLK, DIM), lambda i: (i, 0)),
            pl.BlockSpec((TC_BLK, DIM), lambda i: (nblk - 1 - i, 0)),
        ],
        out_specs=pl.BlockSpec((TC_BLK, DIM), lambda i: (i, 0)),
        out_shape=jax.ShapeDtypeStruct((BATCH, DIM), jnp.float32),
    )(lam.reshape(1, 1), perm, inputs, inputs)

    tgt_i = targets.reshape(BATCH, 1)
    tgt_r = jnp.flip(targets).reshape(BATCH, 1)
    tblk = BATCH // TROWS
    targets_mixed = pl.pallas_call(
        _tc_targets_body,
        grid=(tblk,),
        in_specs=[
            pl.BlockSpec((1, 1), lambda i: (0, 0)),
            pl.BlockSpec((TROWS, 1), lambda i: (i, 0)),
            pl.BlockSpec((TROWS, 1), lambda i: (i, 0)),
        ],
        out_specs=pl.BlockSpec((TROWS, NCLS), lambda i: (i, 0)),
        out_shape=jax.ShapeDtypeStruct((BATCH, NCLS), jnp.float32),
    )(lam.reshape(1, 1), tgt_i, tgt_r)

    return (inputs_mixed, targets_mixed)
